# baseline (speedup 1.0000x reference)
import functools

import jax
import jax.numpy as jnp
from jax import lax
from jax.experimental import pallas as pl
from jax.experimental.pallas import tpu as pltpu
from jax.experimental.pallas import tpu_sc as plsc

N_NODES = 10000
N_EDGES = 320000
D = 128
NC = 2
NS = 16
NW = NC * NS
CH = 79
EPT = CH * 128
EPAD = NW * EPT
NPAD = 10240
RPT = NPAD // NS

_mesh = plsc.VectorSubcoreMesh(
    core_axis_name="c", subcore_axis_name="s", num_cores=NC, num_subcores=NS)

_f32 = jnp.float32
_sc_params = pltpu.CompilerParams(needs_layout_passes=False,
                                  use_tc_tiling_on_sc=False)


def _zero_1d(ref, n):
    def body(i, _):
        ref[pl.ds(i * 16, 16)] = jnp.zeros((16,), _f32)
        return 0
    lax.fori_loop(0, n // 16, body, 0)


@functools.partial(
    pl.kernel,
    out_type=jax.ShapeDtypeStruct((NW * NPAD,), _f32),
    mesh=_mesh,
    scratch_types=[
        pltpu.VMEM((CH, 128), jnp.int32),
        pltpu.VMEM((CH, 128), _f32),
        pltpu.VMEM((NPAD,), _f32),
    ],
    compiler_params=_sc_params,
)
def _deg_kernel(dst_hbm, ew_hbm, out_hbm, dst_v, ew_v, acc):
    c = lax.axis_index("c")
    s = lax.axis_index("s")
    wid = s * NC + c
    pltpu.sync_copy(dst_hbm.at[wid], dst_v)
    pltpu.sync_copy(ew_hbm.at[wid], ew_v)
    _zero_1d(acc, NPAD)

    def body(j, _):
        for k in range(8):
            sl = pl.ds(k * 16, 16)
            d16 = dst_v[j, sl]
            w16 = ew_v[j, sl]
            plsc.addupdate_scatter(acc, [d16], w16)
        return 0
    lax.fori_loop(0, CH, body, 0)
    pltpu.sync_copy(acc, out_hbm.at[pl.ds(wid * NPAD, NPAD)])


_bf16 = jnp.bfloat16


@functools.partial(
    pl.kernel,
    out_type=(
        jax.ShapeDtypeStruct((NW * NPAD,), _f32),
        jax.ShapeDtypeStruct((NC, NPAD, D), _bf16),
    ),
    mesh=_mesh,
    scratch_types=[
        pltpu.VMEM((CH, 128), jnp.int32),
        pltpu.VMEM((CH, 128), jnp.int32),
        pltpu.VMEM((CH, 128), _f32),
        pltpu.VMEM((N_NODES,), _f32),
        pltpu.VMEM((NPAD,), _f32),
        pltpu.VMEM((128, D), _bf16),
        pltpu.VMEM((128, D), _bf16),
        pltpu.VMEM((128, D), _bf16),
        pltpu.SemaphoreType.DMA,
        pltpu.SemaphoreType.DMA,
        pltpu.SemaphoreType.DMA,
        pltpu.SemaphoreType.DMA,
        pltpu.SemaphoreType.DMA,
        pltpu.SemaphoreType.DMA,
        pltpu.VMEM_SHARED((NPAD, D), _bf16),
    ],
    compiler_params=_sc_params,
)
def _edge_kernel(src_hbm, dst_hbm, ew_hbm, dinv_hbm, y_hbm,
                 t_out, agg_out, src_v, dst_v, ew_v, dinv_v, tacc,
                 rows0, rows1, rows2, gs0, gs1, gs2, ss0, ss1, ss2,
                 acc_sh):
    c = lax.axis_index("c")
    s = lax.axis_index("s")
    wid = s * NC + c
    pltpu.sync_copy(src_hbm.at[wid], src_v)
    pltpu.sync_copy(dst_hbm.at[wid], dst_v)
    pltpu.sync_copy(ew_hbm.at[wid], ew_v)
    pltpu.sync_copy(dinv_hbm, dinv_v)
    _zero_1d(tacc, NPAD)

    def tbody(j, _):
        for k in range(8):
            sl = pl.ds(k * 16, 16)
            s16 = src_v[j, sl]
            d16 = dst_v[j, sl]
            w16 = ew_v[j, sl]
            dv = plsc.load_gather(dinv_v, [d16])
            plsc.addupdate_scatter(tacc, [s16], w16 * dv)
        return 0
    lax.fori_loop(0, CH, tbody, 0)
    pltpu.sync_copy(tacc, t_out.at[pl.ds(wid * NPAD, NPAD)])

    R = (rows0, rows1, rows2)
    GS = (gs0, gs1, gs2)
    SS = (ss0, ss1, ss2)

    if True:

        def start_gather(j, b):
            pltpu.async_copy(y_hbm.at[src_v.at[j]], R[b], GS[b])

        def wait_gather(j, b):
            pltpu.make_async_copy(y_hbm.at[src_v.at[j]], R[b], GS[b]).wait()

        def start_scatter(j, b):
            pltpu.async_copy(R[b], acc_sh.at[dst_v.at[j]], SS[b], add=True)

        def wait_scatter(j, b):
            pltpu.make_async_copy(R[b], acc_sh.at[dst_v.at[j]], SS[b]).wait()

        def scale(j, b):
            rb = R[b]

            def sbody(g, _):
                sl0 = pl.ds(g * 16, 16)
                s16 = src_v[j, sl0]
                w16 = ew_v[j, sl0] * plsc.load_gather(dinv_v, [s16])
                for l in range(16):
                    wv = jnp.full((16,), w16[l], dtype=_f32)
                    wb = plsc.pack(wv, wv,
                                   format=plsc.PackFormat.INTERLEAVED)
                    e = g * 16 + l
                    for k in range(D // 32):
                        sl = pl.ds(k * 32, 32)
                        rb[e, sl] = rb[e, sl] * wb
                return 0
            lax.fori_loop(0, 8, sbody, 0)

        def zrow(i, _):
            for k in range(D // 32):
                rows0[i, pl.ds(k * 32, 32)] = jnp.zeros((32,), _bf16)
            return 0
        lax.fori_loop(0, 128, zrow, 0)
        for i in range(RPT // 128):
            pltpu.sync_copy(rows0, acc_sh.at[pl.ds(s * RPT + i * 128, 128)])
        plsc.subcore_barrier()

        start_gather(0, 0)
        start_gather(1, 1)
        wait_gather(0, 0)
        scale(0, 0)
        start_gather(2, 2)
        start_scatter(0, 0)
        wait_gather(1, 1)
        scale(1, 1)
        wait_scatter(0, 0)
        start_gather(3, 0)
        start_scatter(1, 1)
        wait_gather(2, 2)
        scale(2, 2)
        wait_scatter(1, 1)
        start_gather(4, 1)
        start_scatter(2, 2)

        def steady(g, _):
            for b3 in range(3):
                j = 3 * g + b3
                b = b3
                wait_gather(j, b)
                scale(j, b)
                wait_scatter(j - 1, (b + 2) % 3)
                start_gather(j + 2, (b + 2) % 3)
                start_scatter(j, b)
            return 0
        lax.fori_loop(1, 25, steady, 0)

        for j in (75, 76):
            b = j % 3
            wait_gather(j, b)
            scale(j, b)
            wait_scatter(j - 1, (b + 2) % 3)
            start_gather(j + 2, (b + 2) % 3)
            start_scatter(j, b)
        wait_gather(77, 2)
        scale(77, 2)
        wait_scatter(76, 1)
        start_scatter(77, 2)
        wait_gather(78, 0)
        scale(78, 0)
        wait_scatter(77, 2)
        start_scatter(78, 0)
        wait_scatter(78, 0)

        plsc.subcore_barrier()
        pltpu.sync_copy(acc_sh.at[pl.ds(s * RPT, RPT)],
                        agg_out.at[c, pl.ds(s * RPT, RPT)])


def _dense_a1_body(x_ref, w1_ref, y_ref, xw1_ref):
    xw1 = jnp.dot(x_ref[...], w1_ref[...], preferred_element_type=_f32)
    xw1_ref[...] = xw1
    y_ref[...] = xw1.astype(_bf16)


def _dense_a1(x, w1):
    return pl.pallas_call(
        _dense_a1_body,
        out_shape=(
            jax.ShapeDtypeStruct((N_NODES, D), _bf16),
            jax.ShapeDtypeStruct((N_NODES, D), _f32),
        ),
    )(x, w1)


def _dense_a2_body(degp_ref, dinv_ref, dinv2_ref):
    deg = jnp.sum(degp_ref[...], axis=0)[:N_NODES] + 1.0
    dinv = jnp.where(deg > 0, lax.rsqrt(deg), 0.0)
    dinv_ref[...] = dinv
    dinv2_ref[...] = dinv * dinv


def _dense_a2(deg_part):
    return pl.pallas_call(
        _dense_a2_body,
        out_shape=(
            jax.ShapeDtypeStruct((N_NODES,), _f32),
            jax.ShapeDtypeStruct((N_NODES,), _f32),
        ),
    )(deg_part)


def _dense_b_body(aggp_ref, xw1_ref, dinv_ref, dinv2_ref, tp_ref, attr_ref,
                  b1_ref, w2_ref, b2_ref, wm_ref, bm_ref, out_ref):
    dinv = dinv_ref[...]
    dinv2 = dinv2_ref[...]
    agg = (aggp_ref[0].astype(_f32) + aggp_ref[1].astype(_f32))[:N_NODES]
    out1 = dinv[:, None] * agg + dinv2[:, None] * xw1_ref[...] + b1_ref[...][None, :]
    h1 = jnp.maximum(out1, 0.0)
    t = jnp.sum(tp_ref[...], axis=0)[:N_NODES]
    s = dinv * t + dinv2
    v128 = jnp.dot(s[None, :], h1, preferred_element_type=_f32)
    attr = attr_ref[...]
    va = jnp.dot(s[None, :], attr, preferred_element_type=_f32)
    vfull = jnp.concatenate([v128, va], axis=1) * (1.0 / N_NODES)
    mean2 = jnp.dot(vfull, w2_ref[...], preferred_element_type=_f32) + b2_ref[...][None, :]
    mean_attr = jnp.sum(attr, axis=0)[None, :] * (1.0 / N_NODES)
    gv = jnp.concatenate([mean2, mean_attr], axis=1)
    out_ref[...] = jnp.dot(gv, wm_ref[...], preferred_element_type=_f32) + bm_ref[...][None, :]


def _dense_b(agg_part, xw1, dinv, dinv2, t_part, attributes, b1, w2, b2, wm,
             bm):
    return pl.pallas_call(
        _dense_b_body,
        out_shape=jax.ShapeDtypeStruct((1, D), _f32),
    )(agg_part, xw1, dinv, dinv2, t_part, attributes, b1, w2, b2, wm, bm)


def kernel(x, attributes, edge_obj_to_obj, edge_weight, W1, b1, W2, b2, Wm,
           bm):
    src = edge_obj_to_obj[0].astype(jnp.int32)
    dst = edge_obj_to_obj[1].astype(jnp.int32)
    ew = edge_weight.astype(_f32)
    pad = EPAD - N_EDGES
    srcp = jnp.concatenate([src, jnp.zeros((pad,), jnp.int32)]).reshape(
        NW, CH, 128)
    dstp = jnp.concatenate([dst, jnp.zeros((pad,), jnp.int32)]).reshape(
        NW, CH, 128)
    ewp = jnp.concatenate([ew, jnp.zeros((pad,), _f32)]).reshape(NW, CH, 128)

    deg_part = _deg_kernel(dstp, ewp).reshape(NW, NPAD)
    y, xw1 = _dense_a1(x, W1)
    dinv, dinv2 = _dense_a2(deg_part)
    t_part, agg_part = _edge_kernel(srcp, dstp, ewp, dinv, y)
    return _dense_b(agg_part, xw1, dinv, dinv2, t_part.reshape(NW, NPAD),
                    attributes, b1, W2, b2, Wm, bm)


# speedup vs baseline: 1.0232x; 1.0232x over previous
import functools

import jax
import jax.numpy as jnp
from jax import lax
from jax.experimental import pallas as pl
from jax.experimental.pallas import tpu as pltpu
from jax.experimental.pallas import tpu_sc as plsc

N_NODES = 10000
N_EDGES = 320000
D = 128
NC = 2
NS = 16
NW = NC * NS
CH = 79
EPT = CH * 128
EPAD = NW * EPT
NPAD = 10240
RPT = NPAD // NS

_mesh = plsc.VectorSubcoreMesh(
    core_axis_name="c", subcore_axis_name="s", num_cores=NC, num_subcores=NS)

_f32 = jnp.float32
_sc_params = pltpu.CompilerParams(needs_layout_passes=False,
                                  use_tc_tiling_on_sc=False)


def _zero_1d(ref, n):
    def body(i, _):
        ref[pl.ds(i * 16, 16)] = jnp.zeros((16,), _f32)
        return 0
    lax.fori_loop(0, n // 16, body, 0)


@functools.partial(
    pl.kernel,
    out_type=jax.ShapeDtypeStruct((NW * NPAD,), _f32),
    mesh=_mesh,
    scratch_types=[
        pltpu.VMEM((CH, 128), jnp.int32),
        pltpu.VMEM((CH, 128), _f32),
        pltpu.VMEM((NPAD,), _f32),
    ],
    compiler_params=_sc_params,
)
def _deg_kernel(dst_hbm, ew_hbm, out_hbm, dst_v, ew_v, acc):
    c = lax.axis_index("c")
    s = lax.axis_index("s")
    wid = s * NC + c
    pltpu.sync_copy(dst_hbm.at[wid], dst_v)
    pltpu.sync_copy(ew_hbm.at[wid], ew_v)
    _zero_1d(acc, NPAD)

    def body(j, _):
        for k in range(8):
            sl = pl.ds(k * 16, 16)
            d16 = dst_v[j, sl]
            w16 = ew_v[j, sl]
            plsc.addupdate_scatter(acc, [d16], w16)
        return 0
    lax.fori_loop(0, CH, body, 0)
    pltpu.sync_copy(acc, out_hbm.at[pl.ds(wid * NPAD, NPAD)])


_bf16 = jnp.bfloat16


@functools.partial(
    pl.kernel,
    out_type=(
        jax.ShapeDtypeStruct((NW * NPAD,), _f32),
        jax.ShapeDtypeStruct((NC, NPAD, D), _bf16),
    ),
    mesh=_mesh,
    scratch_types=[
        pltpu.VMEM((CH, 128), jnp.int32),
        pltpu.VMEM((CH, 128), jnp.int32),
        pltpu.VMEM((CH, 128), _f32),
        pltpu.VMEM((N_NODES,), _f32),
        pltpu.VMEM((NPAD,), _f32),
        pltpu.VMEM((128, D), _bf16),
        pltpu.VMEM((128, D), _bf16),
        pltpu.VMEM((128, D), _bf16),
        pltpu.SemaphoreType.DMA,
        pltpu.SemaphoreType.DMA,
        pltpu.SemaphoreType.DMA,
        pltpu.SemaphoreType.DMA,
        pltpu.SemaphoreType.DMA,
        pltpu.SemaphoreType.DMA,
        pltpu.VMEM_SHARED((NPAD, D), _bf16),
    ],
    compiler_params=_sc_params,
)
def _edge_kernel(src_hbm, dst_hbm, ew_hbm, dinv_hbm, y_hbm,
                 t_out, agg_out, src_v, dst_v, ew_v, dinv_v, tacc,
                 rows0, rows1, rows2, gs0, gs1, gs2, ss0, ss1, ss2,
                 acc_sh):
    c = lax.axis_index("c")
    s = lax.axis_index("s")
    wid = s * NC + c
    pltpu.sync_copy(src_hbm.at[wid], src_v)
    pltpu.sync_copy(dst_hbm.at[wid], dst_v)
    pltpu.sync_copy(ew_hbm.at[wid], ew_v)
    pltpu.sync_copy(dinv_hbm, dinv_v)
    _zero_1d(tacc, NPAD)

    def tbody(j, _):
        for k in range(8):
            sl = pl.ds(k * 16, 16)
            s16 = src_v[j, sl]
            d16 = dst_v[j, sl]
            w16 = ew_v[j, sl]
            dv = plsc.load_gather(dinv_v, [d16])
            plsc.addupdate_scatter(tacc, [s16], w16 * dv)
        return 0
    lax.fori_loop(0, CH, tbody, 0)
    pltpu.sync_copy(tacc, t_out.at[pl.ds(wid * NPAD, NPAD)])

    R = (rows0, rows1, rows2)
    GS = (gs0, gs1, gs2)
    SS = (ss0, ss1, ss2)

    if True:

        def start_gather(j, b):
            pltpu.async_copy(y_hbm.at[src_v.at[j]], R[b], GS[b])

        def wait_gather(j, b):
            pltpu.make_async_copy(y_hbm.at[src_v.at[j]], R[b], GS[b]).wait()

        def start_scatter(j, b):
            pltpu.async_copy(R[b], acc_sh.at[dst_v.at[j]], SS[b], add=True)

        def wait_scatter(j, b):
            pltpu.make_async_copy(R[b], acc_sh.at[dst_v.at[j]], SS[b]).wait()

        def scale(j, b):
            rb = R[b]

            def sbody(g, _):
                sl0 = pl.ds(g * 16, 16)
                s16 = src_v[j, sl0]
                w16 = ew_v[j, sl0] * plsc.load_gather(dinv_v, [s16])
                for l in range(16):
                    wv = jnp.full((16,), w16[l], dtype=_f32)
                    wb = plsc.pack(wv, wv,
                                   format=plsc.PackFormat.INTERLEAVED)
                    e = g * 16 + l
                    for k in range(D // 32):
                        sl = pl.ds(k * 32, 32)
                        rb[e, sl] = rb[e, sl] * wb
                return 0
            lax.fori_loop(0, 8, sbody, 0)

        def zrow(i, _):
            for k in range(D // 32):
                rows0[i, pl.ds(k * 32, 32)] = jnp.zeros((32,), _bf16)
            return 0
        lax.fori_loop(0, 128, zrow, 0)
        for i in range(RPT // 128):
            pltpu.sync_copy(rows0, acc_sh.at[pl.ds(s * RPT + i * 128, 128)])
        plsc.subcore_barrier()

        start_gather(0, 0)
        start_gather(1, 1)
        wait_gather(0, 0)
        scale(0, 0)
        start_gather(2, 2)
        start_scatter(0, 0)
        wait_gather(1, 1)
        scale(1, 1)
        wait_scatter(0, 0)
        start_gather(3, 0)
        start_scatter(1, 1)
        wait_gather(2, 2)
        scale(2, 2)
        wait_scatter(1, 1)
        start_gather(4, 1)
        start_scatter(2, 2)

        def steady(g, _):
            for b3 in range(3):
                j = 3 * g + b3
                b = b3
                wait_gather(j, b)
                scale(j, b)
                wait_scatter(j - 1, (b + 2) % 3)
                start_gather(j + 2, (b + 2) % 3)
                start_scatter(j, b)
            return 0
        lax.fori_loop(1, 25, steady, 0)

        for j in (75, 76):
            b = j % 3
            wait_gather(j, b)
            scale(j, b)
            wait_scatter(j - 1, (b + 2) % 3)
            start_gather(j + 2, (b + 2) % 3)
            start_scatter(j, b)
        wait_gather(77, 2)
        scale(77, 2)
        wait_scatter(76, 1)
        start_scatter(77, 2)
        wait_gather(78, 0)
        scale(78, 0)
        wait_scatter(77, 2)
        start_scatter(78, 0)
        wait_scatter(78, 0)

        plsc.subcore_barrier()
        pltpu.sync_copy(acc_sh.at[pl.ds(s * RPT, RPT)],
                        agg_out.at[c, pl.ds(s * RPT, RPT)])


def _dense_a_body(degp_ref, x_ref, w1_ref, y_ref, dinv_ref, dinv2_ref):
    deg = jnp.sum(degp_ref[...], axis=0)[:N_NODES] + 1.0
    dinv = jnp.where(deg > 0, lax.rsqrt(deg), 0.0)
    xw1 = jnp.dot(x_ref[...], w1_ref[...], preferred_element_type=_f32)
    y_ref[...] = xw1.astype(_bf16)
    dinv_ref[...] = dinv
    dinv2_ref[...] = dinv * dinv


def _dense_a(deg_part, x, w1):
    return pl.pallas_call(
        _dense_a_body,
        out_shape=(
            jax.ShapeDtypeStruct((N_NODES, D), _bf16),
            jax.ShapeDtypeStruct((N_NODES,), _f32),
            jax.ShapeDtypeStruct((N_NODES,), _f32),
        ),
    )(deg_part, x, w1)


def _dense_b_body(aggp_ref, y_ref, dinv_ref, dinv2_ref, tp_ref, attr_ref,
                  b1_ref, w2_ref, b2_ref, wm_ref, bm_ref, out_ref):
    dinv = dinv_ref[...]
    dinv2 = dinv2_ref[...]
    agg = (aggp_ref[0].astype(_f32) + aggp_ref[1].astype(_f32))[:N_NODES]
    xw1 = y_ref[...].astype(_f32)
    out1 = dinv[:, None] * agg + dinv2[:, None] * xw1 + b1_ref[...][None, :]
    h1 = jnp.maximum(out1, 0.0)
    t = jnp.sum(tp_ref[...], axis=0)[:N_NODES]
    s = dinv * t + dinv2
    v128 = jnp.dot(s[None, :], h1, preferred_element_type=_f32)
    attr = attr_ref[...]
    va = jnp.dot(s[None, :], attr, preferred_element_type=_f32)
    vfull = jnp.concatenate([v128, va], axis=1) * (1.0 / N_NODES)
    mean2 = jnp.dot(vfull, w2_ref[...], preferred_element_type=_f32) + b2_ref[...][None, :]
    mean_attr = jnp.sum(attr, axis=0)[None, :] * (1.0 / N_NODES)
    gv = jnp.concatenate([mean2, mean_attr], axis=1)
    out_ref[...] = jnp.dot(gv, wm_ref[...], preferred_element_type=_f32) + bm_ref[...][None, :]


def _dense_b(agg_part, y, dinv, dinv2, t_part, attributes, b1, w2, b2, wm,
             bm):
    return pl.pallas_call(
        _dense_b_body,
        out_shape=jax.ShapeDtypeStruct((1, D), _f32),
    )(agg_part, y, dinv, dinv2, t_part, attributes, b1, w2, b2, wm, bm)


def kernel(x, attributes, edge_obj_to_obj, edge_weight, W1, b1, W2, b2, Wm,
           bm):
    src = edge_obj_to_obj[0].astype(jnp.int32)
    dst = edge_obj_to_obj[1].astype(jnp.int32)
    ew = edge_weight.astype(_f32)
    pad = EPAD - N_EDGES
    srcp = jnp.concatenate([src, jnp.zeros((pad,), jnp.int32)]).reshape(
        NW, CH, 128)
    dstp = jnp.concatenate([dst, jnp.zeros((pad,), jnp.int32)]).reshape(
        NW, CH, 128)
    ewp = jnp.concatenate([ew, jnp.zeros((pad,), _f32)]).reshape(NW, CH, 128)

    deg_part = _deg_kernel(dstp, ewp).reshape(NW, NPAD)
    y, dinv, dinv2 = _dense_a(deg_part, x, W1)
    t_part, agg_part = _edge_kernel(srcp, dstp, ewp, dinv, y)
    return _dense_b(agg_part, y, dinv, dinv2, t_part.reshape(NW, NPAD),
                    attributes, b1, W2, b2, Wm, bm)


# speedup vs baseline: 1.2842x; 1.2552x over previous
import functools

import jax
import jax.numpy as jnp
from jax import lax
from jax.experimental import pallas as pl
from jax.experimental.pallas import tpu as pltpu
from jax.experimental.pallas import tpu_sc as plsc

N_NODES = 10000
N_EDGES = 320000
D = 128
NC = 2
NS = 16
NW = NC * NS
CH = 79
EPT = CH * 128
EPAD = NW * EPT
NPAD = 10240
RPT = NPAD // NS

_mesh = plsc.VectorSubcoreMesh(
    core_axis_name="c", subcore_axis_name="s", num_cores=NC, num_subcores=NS)

_f32 = jnp.float32
_sc_params = pltpu.CompilerParams(needs_layout_passes=False,
                                  use_tc_tiling_on_sc=False)


def _zero_1d(ref, n):
    def body(i, _):
        ref[pl.ds(i * 16, 16)] = jnp.zeros((16,), _f32)
        return 0
    lax.fori_loop(0, n // 16, body, 0)


@functools.partial(
    pl.kernel,
    out_type=jax.ShapeDtypeStruct((NW * NPAD,), _f32),
    mesh=_mesh,
    scratch_types=[
        pltpu.VMEM((CH, 128), jnp.int32),
        pltpu.VMEM((CH, 128), _f32),
        pltpu.VMEM((NPAD,), _f32),
    ],
    compiler_params=_sc_params,
)
def _deg_kernel(dst_hbm, ew_hbm, out_hbm, dst_v, ew_v, acc):
    c = lax.axis_index("c")
    s = lax.axis_index("s")
    wid = s * NC + c
    pltpu.sync_copy(dst_hbm.at[wid], dst_v)
    pltpu.sync_copy(ew_hbm.at[wid], ew_v)
    _zero_1d(acc, NPAD)

    def body(j, _):
        for k in range(8):
            sl = pl.ds(k * 16, 16)
            d16 = dst_v[j, sl]
            w16 = ew_v[j, sl]
            plsc.addupdate_scatter(acc, [d16], w16)
        return 0
    lax.fori_loop(0, CH, body, 0)
    pltpu.sync_copy(acc, out_hbm.at[pl.ds(wid * NPAD, NPAD)])


_bf16 = jnp.bfloat16


@functools.partial(
    pl.kernel,
    out_type=(
        jax.ShapeDtypeStruct((NW * NPAD,), _f32),
        jax.ShapeDtypeStruct((NC, NPAD, D), _bf16),
    ),
    mesh=_mesh,
    scratch_types=[
        pltpu.VMEM((CH, 128), jnp.int32),
        pltpu.VMEM((CH, 128), jnp.int32),
        pltpu.VMEM((CH, 128), _f32),
        pltpu.VMEM((N_NODES,), _f32),
        pltpu.VMEM((NPAD,), _f32),
        pltpu.VMEM((128, D), _bf16),
        pltpu.VMEM((128, D), _bf16),
        pltpu.VMEM((128, D), _bf16),
        pltpu.SemaphoreType.DMA,
        pltpu.SemaphoreType.DMA,
        pltpu.SemaphoreType.DMA,
        pltpu.SemaphoreType.DMA,
        pltpu.SemaphoreType.DMA,
        pltpu.SemaphoreType.DMA,
        pltpu.VMEM_SHARED((NPAD, D), _bf16),
    ],
    compiler_params=_sc_params,
)
def _edge_kernel(src_hbm, dst_hbm, ew_hbm, dinv_hbm, y_hbm,
                 t_out, agg_out, src_v, dst_v, ew_v, dinv_v, tacc,
                 rows0, rows1, rows2, gs0, gs1, gs2, ss0, ss1, ss2,
                 acc_sh):
    c = lax.axis_index("c")
    s = lax.axis_index("s")
    wid = s * NC + c
    pltpu.sync_copy(src_hbm.at[wid], src_v)
    pltpu.sync_copy(dst_hbm.at[wid], dst_v)
    pltpu.sync_copy(ew_hbm.at[wid], ew_v)
    pltpu.sync_copy(dinv_hbm, dinv_v)
    _zero_1d(tacc, NPAD)

    def tbody(j, _):
        for k in range(8):
            sl = pl.ds(k * 16, 16)
            s16 = src_v[j, sl]
            d16 = dst_v[j, sl]
            w16 = ew_v[j, sl]
            dv = plsc.load_gather(dinv_v, [d16])
            plsc.addupdate_scatter(tacc, [s16], w16 * dv)
        return 0
    lax.fori_loop(0, CH, tbody, 0)
    pltpu.sync_copy(tacc, t_out.at[pl.ds(wid * NPAD, NPAD)])

    R = (rows0, rows1, rows2)
    GS = (gs0, gs1, gs2)
    SS = (ss0, ss1, ss2)

    if True:

        def start_gather(j, b):
            pltpu.async_copy(y_hbm.at[src_v.at[j]], R[b], GS[b])

        def wait_gather(j, b):
            pltpu.make_async_copy(y_hbm.at[src_v.at[j]], R[b], GS[b]).wait()

        def start_scatter(j, b):
            pltpu.async_copy(R[b], acc_sh.at[dst_v.at[j]], SS[b], add=True)

        def wait_scatter(j, b):
            pltpu.make_async_copy(R[b], acc_sh.at[dst_v.at[j]], SS[b]).wait()

        def scale(j, b):
            rb = R[b]

            def sbody(g, _):
                w16 = ew_v[j, pl.ds(g * 16, 16)]
                for l in range(16):
                    wv = jnp.full((16,), w16[l], dtype=_f32)
                    wb = plsc.pack(wv, wv,
                                   format=plsc.PackFormat.INTERLEAVED)
                    e = g * 16 + l
                    for k in range(D // 32):
                        sl = pl.ds(k * 32, 32)
                        rb[e, sl] = rb[e, sl] * wb
                return 0
            lax.fori_loop(0, 8, sbody, 0)

        def zrow(i, _):
            for k in range(D // 32):
                rows0[i, pl.ds(k * 32, 32)] = jnp.zeros((32,), _bf16)
            return 0
        lax.fori_loop(0, 128, zrow, 0)
        for i in range(RPT // 128):
            pltpu.sync_copy(rows0, acc_sh.at[pl.ds(s * RPT + i * 128, 128)])
        plsc.subcore_barrier()

        start_gather(0, 0)
        start_gather(1, 1)
        wait_gather(0, 0)
        scale(0, 0)
        start_gather(2, 2)
        start_scatter(0, 0)
        wait_gather(1, 1)
        scale(1, 1)
        wait_scatter(0, 0)
        start_gather(3, 0)
        start_scatter(1, 1)
        wait_gather(2, 2)
        scale(2, 2)
        wait_scatter(1, 1)
        start_gather(4, 1)
        start_scatter(2, 2)

        def steady(g, _):
            for b3 in range(3):
                j = 3 * g + b3
                b = b3
                wait_gather(j, b)
                scale(j, b)
                wait_scatter(j - 1, (b + 2) % 3)
                start_gather(j + 2, (b + 2) % 3)
                start_scatter(j, b)
            return 0
        lax.fori_loop(1, 25, steady, 0)

        for j in (75, 76):
            b = j % 3
            wait_gather(j, b)
            scale(j, b)
            wait_scatter(j - 1, (b + 2) % 3)
            start_gather(j + 2, (b + 2) % 3)
            start_scatter(j, b)
        wait_gather(77, 2)
        scale(77, 2)
        wait_scatter(76, 1)
        start_scatter(77, 2)
        wait_gather(78, 0)
        scale(78, 0)
        wait_scatter(77, 2)
        start_scatter(78, 0)
        wait_scatter(78, 0)

        plsc.subcore_barrier()
        pltpu.sync_copy(acc_sh.at[pl.ds(s * RPT, RPT)],
                        agg_out.at[c, pl.ds(s * RPT, RPT)])


def _dense_a_body(degp_ref, x_ref, w1_ref, y_ref, dinv_ref, dinv2_ref):
    deg = jnp.sum(degp_ref[...], axis=0)[:N_NODES] + 1.0
    dinv = jnp.where(deg > 0, lax.rsqrt(deg), 0.0)
    xw1 = jnp.dot(x_ref[...], w1_ref[...], preferred_element_type=_f32)
    y_ref[...] = (dinv[:, None] * xw1).astype(_bf16)
    dinv_ref[...] = dinv
    dinv2_ref[...] = dinv * dinv


def _dense_a(deg_part, x, w1):
    return pl.pallas_call(
        _dense_a_body,
        out_shape=(
            jax.ShapeDtypeStruct((N_NODES, D), _bf16),
            jax.ShapeDtypeStruct((N_NODES,), _f32),
            jax.ShapeDtypeStruct((N_NODES,), _f32),
        ),
    )(deg_part, x, w1)


def _dense_b_body(aggp_ref, y_ref, dinv_ref, dinv2_ref, tp_ref, attr_ref,
                  b1_ref, w2_ref, b2_ref, wm_ref, bm_ref, out_ref):
    dinv = dinv_ref[...]
    dinv2 = dinv2_ref[...]
    agg = (aggp_ref[0].astype(_f32) + aggp_ref[1].astype(_f32))[:N_NODES]
    agg = agg + y_ref[...].astype(_f32)
    out1 = dinv[:, None] * agg + b1_ref[...][None, :]
    h1 = jnp.maximum(out1, 0.0)
    t = jnp.sum(tp_ref[...], axis=0)[:N_NODES]
    s = dinv * t + dinv2
    v128 = jnp.dot(s[None, :], h1, preferred_element_type=_f32)
    attr = attr_ref[...]
    va = jnp.dot(s[None, :], attr, preferred_element_type=_f32)
    vfull = jnp.concatenate([v128, va], axis=1) * (1.0 / N_NODES)
    mean2 = jnp.dot(vfull, w2_ref[...], preferred_element_type=_f32) + b2_ref[...][None, :]
    mean_attr = jnp.sum(attr, axis=0)[None, :] * (1.0 / N_NODES)
    gv = jnp.concatenate([mean2, mean_attr], axis=1)
    out_ref[...] = jnp.dot(gv, wm_ref[...], preferred_element_type=_f32) + bm_ref[...][None, :]


def _dense_b(agg_part, y, dinv, dinv2, t_part, attributes, b1, w2, b2, wm,
             bm):
    return pl.pallas_call(
        _dense_b_body,
        out_shape=jax.ShapeDtypeStruct((1, D), _f32),
    )(agg_part, y, dinv, dinv2, t_part, attributes, b1, w2, b2, wm, bm)


def kernel(x, attributes, edge_obj_to_obj, edge_weight, W1, b1, W2, b2, Wm,
           bm):
    src = edge_obj_to_obj[0].astype(jnp.int32)
    dst = edge_obj_to_obj[1].astype(jnp.int32)
    ew = edge_weight.astype(_f32)
    pad = EPAD - N_EDGES
    srcp = jnp.concatenate([src, jnp.zeros((pad,), jnp.int32)]).reshape(
        NW, CH, 128)
    dstp = jnp.concatenate([dst, jnp.zeros((pad,), jnp.int32)]).reshape(
        NW, CH, 128)
    ewp = jnp.concatenate([ew, jnp.zeros((pad,), _f32)]).reshape(NW, CH, 128)

    deg_part = _deg_kernel(dstp, ewp).reshape(NW, NPAD)
    y, dinv, dinv2 = _dense_a(deg_part, x, W1)
    t_part, agg_part = _edge_kernel(srcp, dstp, ewp, dinv, y)
    return _dense_b(agg_part, y, dinv, dinv2, t_part.reshape(NW, NPAD),
                    attributes, b1, W2, b2, Wm, bm)


# speedup vs baseline: 1.2869x; 1.0021x over previous
import functools

import jax
import jax.numpy as jnp
from jax import lax
from jax.experimental import pallas as pl
from jax.experimental.pallas import tpu as pltpu
from jax.experimental.pallas import tpu_sc as plsc

N_NODES = 10000
N_EDGES = 320000
D = 128
NC = 2
NS = 16
NW = NC * NS
CH = 79
EPT = CH * 128
EPAD = NW * EPT
NPAD = 10240
RPT = NPAD // NS

_mesh = plsc.VectorSubcoreMesh(
    core_axis_name="c", subcore_axis_name="s", num_cores=NC, num_subcores=NS)

_f32 = jnp.float32
_sc_params = pltpu.CompilerParams(needs_layout_passes=False,
                                  use_tc_tiling_on_sc=False)


def _zero_1d(ref, n):
    def body(i, _):
        ref[pl.ds(i * 16, 16)] = jnp.zeros((16,), _f32)
        return 0
    lax.fori_loop(0, n // 16, body, 0)


@functools.partial(
    pl.kernel,
    out_type=jax.ShapeDtypeStruct((NW * NPAD,), _f32),
    mesh=_mesh,
    scratch_types=[
        pltpu.VMEM((CH, 128), jnp.int32),
        pltpu.VMEM((CH, 128), _f32),
        pltpu.VMEM((NPAD,), _f32),
    ],
    compiler_params=_sc_params,
)
def _deg_kernel(dst_hbm, ew_hbm, out_hbm, dst_v, ew_v, acc):
    c = lax.axis_index("c")
    s = lax.axis_index("s")
    wid = s * NC + c
    pltpu.sync_copy(dst_hbm.at[wid], dst_v)
    pltpu.sync_copy(ew_hbm.at[wid], ew_v)
    _zero_1d(acc, NPAD)

    def body(j, _):
        for k in range(8):
            sl = pl.ds(k * 16, 16)
            d16 = dst_v[j, sl]
            w16 = ew_v[j, sl]
            plsc.addupdate_scatter(acc, [d16], w16)
        return 0
    lax.fori_loop(0, CH, body, 0)
    pltpu.sync_copy(acc, out_hbm.at[pl.ds(wid * NPAD, NPAD)])


_bf16 = jnp.bfloat16


@functools.partial(
    pl.kernel,
    out_type=(
        jax.ShapeDtypeStruct((NW * NPAD,), _f32),
        jax.ShapeDtypeStruct((NC, NPAD, D), _bf16),
    ),
    mesh=_mesh,
    scratch_types=[
        pltpu.VMEM((CH, 128), jnp.int32),
        pltpu.VMEM((CH, 128), jnp.int32),
        pltpu.VMEM((CH, 128), _f32),
        pltpu.VMEM((N_NODES,), _f32),
        pltpu.VMEM((NPAD,), _f32),
        pltpu.VMEM((128, D), _bf16),
        pltpu.VMEM((128, D), _bf16),
        pltpu.VMEM((128, D), _bf16),
        pltpu.SemaphoreType.DMA,
        pltpu.SemaphoreType.DMA,
        pltpu.SemaphoreType.DMA,
        pltpu.SemaphoreType.DMA,
        pltpu.SemaphoreType.DMA,
        pltpu.SemaphoreType.DMA,
        pltpu.VMEM_SHARED((NPAD, D), _bf16),
    ],
    compiler_params=_sc_params,
)
def _edge_kernel(src_hbm, dst_hbm, ew_hbm, dinv_hbm, y_hbm,
                 t_out, agg_out, src_v, dst_v, ew_v, dinv_v, tacc,
                 rows0, rows1, rows2, gs0, gs1, gs2, ss0, ss1, ss2,
                 acc_sh):
    c = lax.axis_index("c")
    s = lax.axis_index("s")
    wid = s * NC + c
    pltpu.sync_copy(src_hbm.at[wid], src_v)
    pltpu.sync_copy(dst_hbm.at[wid], dst_v)
    pltpu.sync_copy(ew_hbm.at[wid], ew_v)
    pltpu.sync_copy(dinv_hbm, dinv_v)
    _zero_1d(tacc, NPAD)

    R = (rows0, rows1, rows2)
    GS = (gs0, gs1, gs2)
    SS = (ss0, ss1, ss2)

    if True:

        def start_gather(j, b):
            pltpu.async_copy(y_hbm.at[src_v.at[j]], R[b], GS[b])

        def wait_gather(j, b):
            pltpu.make_async_copy(y_hbm.at[src_v.at[j]], R[b], GS[b]).wait()

        def start_scatter(j, b):
            pltpu.async_copy(R[b], acc_sh.at[dst_v.at[j]], SS[b], add=True)

        def wait_scatter(j, b):
            pltpu.make_async_copy(R[b], acc_sh.at[dst_v.at[j]], SS[b]).wait()

        def scale(j, b):
            rb = R[b]

            def sbody(g, _):
                w16 = ew_v[j, pl.ds(g * 16, 16)]
                for l in range(16):
                    wv = jnp.full((16,), w16[l], dtype=_f32)
                    wb = plsc.pack(wv, wv,
                                   format=plsc.PackFormat.INTERLEAVED)
                    e = g * 16 + l
                    for k in range(D // 32):
                        sl = pl.ds(k * 32, 32)
                        rb[e, sl] = rb[e, sl] * wb
                return 0
            lax.fori_loop(0, 8, sbody, 0)

        def zrow(i, _):
            for k in range(D // 32):
                rows0[i, pl.ds(k * 32, 32)] = jnp.zeros((32,), _bf16)
            return 0
        lax.fori_loop(0, 128, zrow, 0)
        for i in range(RPT // 128):
            pltpu.sync_copy(rows0, acc_sh.at[pl.ds(s * RPT + i * 128, 128)])
        plsc.subcore_barrier()

        start_gather(0, 0)
        start_gather(1, 1)

        def tbody(j, _):
            for k in range(8):
                sl = pl.ds(k * 16, 16)
                s16 = src_v[j, sl]
                d16 = dst_v[j, sl]
                w16 = ew_v[j, sl]
                dv = plsc.load_gather(dinv_v, [d16])
                plsc.addupdate_scatter(tacc, [s16], w16 * dv)
            return 0
        lax.fori_loop(0, CH, tbody, 0)
        pltpu.sync_copy(tacc, t_out.at[pl.ds(wid * NPAD, NPAD)])

        wait_gather(0, 0)
        scale(0, 0)
        start_gather(2, 2)
        start_scatter(0, 0)
        wait_gather(1, 1)
        scale(1, 1)
        wait_scatter(0, 0)
        start_gather(3, 0)
        start_scatter(1, 1)
        wait_gather(2, 2)
        scale(2, 2)
        wait_scatter(1, 1)
        start_gather(4, 1)
        start_scatter(2, 2)

        def steady(g, _):
            for b3 in range(3):
                j = 3 * g + b3
                b = b3
                wait_gather(j, b)
                scale(j, b)
                wait_scatter(j - 1, (b + 2) % 3)
                start_gather(j + 2, (b + 2) % 3)
                start_scatter(j, b)
            return 0
        lax.fori_loop(1, 25, steady, 0)

        for j in (75, 76):
            b = j % 3
            wait_gather(j, b)
            scale(j, b)
            wait_scatter(j - 1, (b + 2) % 3)
            start_gather(j + 2, (b + 2) % 3)
            start_scatter(j, b)
        wait_gather(77, 2)
        scale(77, 2)
        wait_scatter(76, 1)
        start_scatter(77, 2)
        wait_gather(78, 0)
        scale(78, 0)
        wait_scatter(77, 2)
        start_scatter(78, 0)
        wait_scatter(78, 0)

        plsc.subcore_barrier()
        pltpu.sync_copy(acc_sh.at[pl.ds(s * RPT, RPT)],
                        agg_out.at[c, pl.ds(s * RPT, RPT)])


def _dense_a_body(degp_ref, x_ref, w1_ref, y_ref, dinv_ref, dinv2_ref):
    deg = jnp.sum(degp_ref[...], axis=0)[:N_NODES] + 1.0
    dinv = jnp.where(deg > 0, lax.rsqrt(deg), 0.0)
    xw1 = jnp.dot(x_ref[...], w1_ref[...], preferred_element_type=_f32)
    y_ref[...] = (dinv[:, None] * xw1).astype(_bf16)
    dinv_ref[...] = dinv
    dinv2_ref[...] = dinv * dinv


def _dense_a(deg_part, x, w1):
    return pl.pallas_call(
        _dense_a_body,
        out_shape=(
            jax.ShapeDtypeStruct((N_NODES, D), _bf16),
            jax.ShapeDtypeStruct((N_NODES,), _f32),
            jax.ShapeDtypeStruct((N_NODES,), _f32),
        ),
    )(deg_part, x, w1)


def _dense_b_body(aggp_ref, y_ref, dinv_ref, dinv2_ref, tp_ref, attr_ref,
                  b1_ref, w2_ref, b2_ref, wm_ref, bm_ref, out_ref):
    dinv = dinv_ref[...]
    dinv2 = dinv2_ref[...]
    agg = (aggp_ref[0].astype(_f32) + aggp_ref[1].astype(_f32))[:N_NODES]
    agg = agg + y_ref[...].astype(_f32)
    out1 = dinv[:, None] * agg + b1_ref[...][None, :]
    h1 = jnp.maximum(out1, 0.0)
    t = jnp.sum(tp_ref[...], axis=0)[:N_NODES]
    s = dinv * t + dinv2
    v128 = jnp.dot(s[None, :], h1, preferred_element_type=_f32)
    attr = attr_ref[...]
    va = jnp.dot(s[None, :], attr, preferred_element_type=_f32)
    vfull = jnp.concatenate([v128, va], axis=1) * (1.0 / N_NODES)
    mean2 = jnp.dot(vfull, w2_ref[...], preferred_element_type=_f32) + b2_ref[...][None, :]
    mean_attr = jnp.sum(attr, axis=0)[None, :] * (1.0 / N_NODES)
    gv = jnp.concatenate([mean2, mean_attr], axis=1)
    out_ref[...] = jnp.dot(gv, wm_ref[...], preferred_element_type=_f32) + bm_ref[...][None, :]


def _dense_b(agg_part, y, dinv, dinv2, t_part, attributes, b1, w2, b2, wm,
             bm):
    return pl.pallas_call(
        _dense_b_body,
        out_shape=jax.ShapeDtypeStruct((1, D), _f32),
    )(agg_part, y, dinv, dinv2, t_part, attributes, b1, w2, b2, wm, bm)


def kernel(x, attributes, edge_obj_to_obj, edge_weight, W1, b1, W2, b2, Wm,
           bm):
    src = edge_obj_to_obj[0].astype(jnp.int32)
    dst = edge_obj_to_obj[1].astype(jnp.int32)
    ew = edge_weight.astype(_f32)
    pad = EPAD - N_EDGES
    srcp = jnp.concatenate([src, jnp.zeros((pad,), jnp.int32)]).reshape(
        NW, CH, 128)
    dstp = jnp.concatenate([dst, jnp.zeros((pad,), jnp.int32)]).reshape(
        NW, CH, 128)
    ewp = jnp.concatenate([ew, jnp.zeros((pad,), _f32)]).reshape(NW, CH, 128)

    deg_part = _deg_kernel(dstp, ewp).reshape(NW, NPAD)
    y, dinv, dinv2 = _dense_a(deg_part, x, W1)
    t_part, agg_part = _edge_kernel(srcp, dstp, ewp, dinv, y)
    return _dense_b(agg_part, y, dinv, dinv2, t_part.reshape(NW, NPAD),
                    attributes, b1, W2, b2, Wm, bm)
